# Initial kernel scaffold; baseline (speedup 1.0000x reference)
import jax
import jax.numpy as jnp
from jax.experimental import pallas as pl


def kernel(embeddings, target):
    raise NotImplementedError("write your pallas kernel here")


# speedup vs baseline: 323.0574x; 323.0574x over previous
import functools

import jax
import jax.numpy as jnp
from jax import lax
from jax.experimental import pallas as pl
from jax.experimental.pallas import tpu as pltpu
from jax.experimental.pallas import tpu_sc as plsc

_B = 1024
_D = 64
_MARGIN = 1.0
_N_PAIRS = _B * (_B - 1) // 2

_L = 16
_NW = 16
_ROWS = _B // _NW
_DP = 80
_CP = 128
_ZROWS = _CP // _NW


def _sc_pos_kernel(emb_hbm, tgt_hbm, out_hbm, rows_v, tgt_v, aug_v, zbuf_v,
                   acc_sh, acc_v, out_v, sem):
    wid = lax.axis_index("s")
    base = wid * _ROWS

    zvec = jnp.zeros((_L,), jnp.float32)
    for r in range(_ZROWS):
        for c in range(_DP // _L):
            zbuf_v[r, pl.ds(c * _L, _L)] = zvec
    pltpu.sync_copy(zbuf_v, acc_sh.at[pl.ds(wid * _ZROWS, _ZROWS)])

    pltpu.sync_copy(emb_hbm.at[pl.ds(base, _ROWS)], rows_v)
    pltpu.sync_copy(tgt_hbm.at[pl.ds(base, _ROWS)], tgt_v)

    lane = lax.iota(jnp.int32, _L)

    def build_row(r, _):
        c0 = rows_v[r, pl.ds(0, _L)]
        c1 = rows_v[r, pl.ds(_L, _L)]
        c2 = rows_v[r, pl.ds(2 * _L, _L)]
        c3 = rows_v[r, pl.ds(3 * _L, _L)]
        aug_v[r, pl.ds(0, _L)] = c0
        aug_v[r, pl.ds(_L, _L)] = c1
        aug_v[r, pl.ds(2 * _L, _L)] = c2
        aug_v[r, pl.ds(3 * _L, _L)] = c3
        sumsq = jnp.sum(c0 * c0 + c1 * c1 + c2 * c2 + c3 * c3)
        tail = jnp.where(lane == 0, sumsq,
                         jnp.where(lane == 1, jnp.float32(1.0),
                                   jnp.float32(0.0)))
        aug_v[r, pl.ds(4 * _L, _L)] = tail
        return 0

    lax.fori_loop(0, _ROWS, build_row, 0)

    plsc.subcore_barrier()
    pltpu.sync_copy(aug_v, acc_sh.at[tgt_v], add=True)
    plsc.subcore_barrier()

    @pl.when(wid == 0)
    def _():
        pltpu.sync_copy(acc_sh, acc_v)

        def class_term(c, tot):
            v0 = acc_v[c, pl.ds(0, _L)]
            v1 = acc_v[c, pl.ds(_L, _L)]
            v2 = acc_v[c, pl.ds(2 * _L, _L)]
            v3 = acc_v[c, pl.ds(3 * _L, _L)]
            ssum = jnp.sum(v0 * v0 + v1 * v1 + v2 * v2 + v3 * v3)
            tail = acc_v[c, pl.ds(4 * _L, _L)]
            m = jnp.sum(jnp.where(lane == 0, tail, jnp.float32(0.0)))
            n = jnp.sum(jnp.where(lane == 1, tail, jnp.float32(0.0)))
            return tot + (n * m - ssum)

        total = lax.fori_loop(0, _CP, class_term, jnp.float32(0.0))
        out_v[...] = jnp.broadcast_to(
            total * jnp.float32(1.0 / _N_PAIRS), (_L,))
        pltpu.sync_copy(out_v, out_hbm)


_sc_pos = functools.partial(
    pl.kernel,
    out_type=jax.ShapeDtypeStruct((_L,), jnp.float32),
    mesh=plsc.VectorSubcoreMesh(
        core_axis_name="c", subcore_axis_name="s", num_cores=1),
    scratch_types=[
        pltpu.VMEM((_ROWS, _D), jnp.float32),
        pltpu.VMEM((_ROWS,), jnp.int32),
        pltpu.VMEM((_ROWS, _DP), jnp.float32),
        pltpu.VMEM((_ZROWS, _DP), jnp.float32),
        pltpu.VMEM_SHARED((_CP, _DP), jnp.float32),
        pltpu.VMEM((_CP, _DP), jnp.float32),
        pltpu.VMEM((_L,), jnp.float32),
        pltpu.SemaphoreType.DMA,
    ],
    compiler_params=pltpu.CompilerParams(needs_layout_passes=False),
)(_sc_pos_kernel)


def _tc_neg_kernel(e_ref, t_ref, out_ref):
    e = e_ref[...]
    t = t_ref[...]
    g = lax.dot_general(
        e, e, (((1,), (1,)), ((), ())), preferred_element_type=jnp.float32
    )
    nrm = jnp.sum(e * e, axis=1, keepdims=True)
    d2 = jnp.maximum(nrm + nrm.T - 2.0 * g, 0.0)
    neg = jnp.maximum(_MARGIN - jnp.sqrt(d2 + 1e-6), 0.0)
    diff = t != t.reshape(1, _B)
    total = 0.5 * jnp.sum(jnp.where(diff, neg * neg, 0.0))
    out_ref[...] = (total / jnp.float32(_N_PAIRS)).reshape(1, 1)


def kernel(embeddings, target):
    pos = _sc_pos(embeddings, target)
    neg = pl.pallas_call(
        _tc_neg_kernel,
        out_shape=jax.ShapeDtypeStruct((1, 1), jnp.float32),
    )(embeddings, target.reshape(_B, 1))
    return neg[0, 0] + pos[0]


# speedup vs baseline: 323.1144x; 1.0002x over previous
import functools

import jax
import jax.numpy as jnp
from jax import lax
from jax.experimental import pallas as pl
from jax.experimental.pallas import tpu as pltpu
from jax.experimental.pallas import tpu_sc as plsc

_B = 1024
_D = 64
_MARGIN = 1.0
_N_PAIRS = _B * (_B - 1) // 2

_L = 16
_NW = 16
_ROWS = _B // _NW
_DP = 80
_CP = 128
_ZROWS = _CP // _NW


def _sc_pos_kernel(emb_hbm, tgt_hbm, out_hbm, rows_v, tgt_v, aug_v, zbuf_v,
                   acc_sh, accw_v, part_sh, pmat_v, out_v, sem):
    wid = lax.axis_index("s")
    base = wid * _ROWS

    zvec = jnp.zeros((_L,), jnp.float32)
    for r in range(_ZROWS):
        for c in range(_DP // _L):
            zbuf_v[r, pl.ds(c * _L, _L)] = zvec
    pltpu.sync_copy(zbuf_v, acc_sh.at[pl.ds(wid * _ZROWS, _ZROWS)])

    pltpu.sync_copy(emb_hbm.at[pl.ds(base, _ROWS)], rows_v)
    pltpu.sync_copy(tgt_hbm.at[pl.ds(base, _ROWS)], tgt_v)

    lane = lax.iota(jnp.int32, _L)
    for r in range(_ROWS):
        c0 = rows_v[r, pl.ds(0, _L)]
        c1 = rows_v[r, pl.ds(_L, _L)]
        c2 = rows_v[r, pl.ds(2 * _L, _L)]
        c3 = rows_v[r, pl.ds(3 * _L, _L)]
        aug_v[r, pl.ds(0, _L)] = c0
        aug_v[r, pl.ds(_L, _L)] = c1
        aug_v[r, pl.ds(2 * _L, _L)] = c2
        aug_v[r, pl.ds(3 * _L, _L)] = c3
        sumsq = jnp.sum(c0 * c0 + c1 * c1 + c2 * c2 + c3 * c3)
        tail = jnp.where(lane == 0, sumsq,
                         jnp.where(lane == 1, jnp.float32(1.0),
                                   jnp.float32(0.0)))
        aug_v[r, pl.ds(4 * _L, _L)] = tail

    plsc.subcore_barrier()
    pltpu.sync_copy(aug_v, acc_sh.at[tgt_v], add=True)
    plsc.subcore_barrier()

    cpw = _CP // _NW
    pltpu.sync_copy(acc_sh.at[pl.ds(wid * cpw, cpw)], accw_v)
    total = jnp.float32(0.0)
    for c in range(cpw):
        v0 = accw_v[c, pl.ds(0, _L)]
        v1 = accw_v[c, pl.ds(_L, _L)]
        v2 = accw_v[c, pl.ds(2 * _L, _L)]
        v3 = accw_v[c, pl.ds(3 * _L, _L)]
        ssum = jnp.sum(v0 * v0 + v1 * v1 + v2 * v2 + v3 * v3)
        tail = accw_v[c, pl.ds(4 * _L, _L)]
        m = jnp.sum(jnp.where(lane == 0, tail, jnp.float32(0.0)))
        n = jnp.sum(jnp.where(lane == 1, tail, jnp.float32(0.0)))
        total = total + (n * m - ssum)
    out_v[...] = jnp.broadcast_to(total, (_L,))
    pltpu.sync_copy(out_v, part_sh.at[wid])
    plsc.subcore_barrier()

    @pl.when(wid == 0)
    def _():
        pltpu.sync_copy(part_sh, pmat_v)
        tot = pmat_v[0, pl.ds(0, _L)]
        for w in range(1, _NW):
            tot = tot + pmat_v[w, pl.ds(0, _L)]
        out_v[...] = tot * jnp.float32(1.0 / _N_PAIRS)
        pltpu.sync_copy(out_v, out_hbm)


_sc_pos = functools.partial(
    pl.kernel,
    out_type=jax.ShapeDtypeStruct((_L,), jnp.float32),
    mesh=plsc.VectorSubcoreMesh(
        core_axis_name="c", subcore_axis_name="s", num_cores=1),
    scratch_types=[
        pltpu.VMEM((_ROWS, _D), jnp.float32),
        pltpu.VMEM((_ROWS,), jnp.int32),
        pltpu.VMEM((_ROWS, _DP), jnp.float32),
        pltpu.VMEM((_ZROWS, _DP), jnp.float32),
        pltpu.VMEM_SHARED((_CP, _DP), jnp.float32),
        pltpu.VMEM((_CP // _NW, _DP), jnp.float32),
        pltpu.VMEM_SHARED((_NW, _L), jnp.float32),
        pltpu.VMEM((_NW, _L), jnp.float32),
        pltpu.VMEM((_L,), jnp.float32),
        pltpu.SemaphoreType.DMA,
    ],
    compiler_params=pltpu.CompilerParams(needs_layout_passes=False),
)(_sc_pos_kernel)


def _tc_neg_kernel(e_ref, t_ref, out_ref):
    e = e_ref[...]
    t = t_ref[...]
    g = lax.dot_general(
        e, e, (((1,), (1,)), ((), ())), preferred_element_type=jnp.float32
    )
    nrm = jnp.sum(e * e, axis=1, keepdims=True)
    d2 = jnp.maximum(nrm + nrm.T - 2.0 * g, 0.0)
    neg = jnp.maximum(_MARGIN - jnp.sqrt(d2 + 1e-6), 0.0)
    diff = t != t.reshape(1, _B)
    total = 0.5 * jnp.sum(jnp.where(diff, neg * neg, 0.0))
    out_ref[...] = (total / jnp.float32(_N_PAIRS)).reshape(1, 1)


def kernel(embeddings, target):
    pos = _sc_pos(embeddings, target)
    neg = pl.pallas_call(
        _tc_neg_kernel,
        out_shape=jax.ShapeDtypeStruct((1, 1), jnp.float32),
    )(embeddings, target.reshape(_B, 1))
    return neg[0, 0] + pos[0]
